# baseline (speedup 1.0000x reference)
import functools

import jax
import jax.numpy as jnp
from jax import lax
from jax.experimental import pallas as pl
from jax.experimental.pallas import tpu as pltpu
from jax.experimental.pallas import tpu_sc as plsc

D = 64
F = 128

NC = 2
NS = 16
NW = NC * NS

IDX_ROW = 128
SUBS = 4
CHUNK = IDX_ROW * SUBS
HALF = CHUNK // 2


def _gather_body(idx_hbm, table_hbm, out_hbm, raw_v, idx_v, rows_v, gsem0,
                 gsem1, osem0, osem1, *, n_chunks):
    wid = lax.axis_index("s") * NC + lax.axis_index("c")
    base_tok = wid * (n_chunks * CHUNK)
    ar = jax.lax.iota(jnp.int32, 16)
    pbase = (ar // 2) + (ar % 2) * HALF
    gsems = [gsem0, gsem1]
    osems = [osem0, osem1]

    def stage(c, b):
        tok_off = base_tok + c * CHUNK
        pltpu.sync_copy(idx_hbm.at[pl.ds(tok_off, CHUNK)], raw_v.at[b])
        for g in range(CHUNK // 16):
            vals = plsc.load_gather(raw_v.at[b], [pbase + (8 * g)])
            idx_v[b, (g * 16) // IDX_ROW, pl.ds((g * 16) % IDX_ROW, 16)] = vals
        for j in range(SUBS):
            pltpu.async_copy(
                table_hbm.at[idx_v.at[b].at[j]],
                rows_v.at[b].at[pl.ds(j * IDX_ROW, IDX_ROW)],
                gsems[b],
            )

    def drain_gathers(b):
        for j in range(SUBS):
            pltpu.make_async_copy(
                table_hbm.at[idx_v.at[b].at[j]],
                rows_v.at[b].at[pl.ds(j * IDX_ROW, IDX_ROW)],
                gsems[b],
            ).wait()

    def flush(c, b):
        tok_off = base_tok + c * CHUNK
        pltpu.async_copy(
            rows_v.at[b], out_hbm.at[pl.ds(tok_off, CHUNK)], osems[b]
        )

    def drain_flush(c, b):
        tok_off = base_tok + c * CHUNK
        pltpu.make_async_copy(
            rows_v.at[b], out_hbm.at[pl.ds(tok_off, CHUNK)], osems[b]
        ).wait()

    stage(0, 0)
    np2 = n_chunks // 2

    def body(p, carry):
        c0 = 2 * p

        stage(c0 + 1, 1)
        drain_gathers(0)
        flush(c0, 0)
        drain_gathers(1)
        drain_flush(c0, 0)

        @pl.when(p + 1 < np2)
        def _():
            stage(c0 + 2, 0)

        flush(c0 + 1, 1)
        drain_flush(c0 + 1, 1)
        return carry

    lax.fori_loop(0, np2, body, 0)


@functools.lru_cache(maxsize=None)
def _make_gather(tok):
    n_chunks = tok // (NW * CHUNK)
    mesh = plsc.VectorSubcoreMesh(core_axis_name="c", subcore_axis_name="s")
    return pl.kernel(
        functools.partial(_gather_body, n_chunks=n_chunks),
        out_type=jax.ShapeDtypeStruct((tok, D), jnp.float32),
        mesh=mesh,
        scratch_types=[
            pltpu.VMEM((2, CHUNK), jnp.int32),
            pltpu.VMEM((2, SUBS, IDX_ROW), jnp.int32),
            pltpu.VMEM((2, CHUNK, D), jnp.float32),
            pltpu.SemaphoreType.DMA,
            pltpu.SemaphoreType.DMA,
            pltpu.SemaphoreType.DMA,
            pltpu.SemaphoreType.DMA,
        ],
        compiler_params=pltpu.CompilerParams(
            use_tc_tiling_on_sc=False, needs_layout_passes=False
        ),
    )


RB = 8192


NSEG = 5


def _dense_body_first(x_ref, w_ref, b_ref, o_ref):
    y = jnp.dot(
        x_ref[...].astype(jnp.bfloat16),
        w_ref[...],
        preferred_element_type=jnp.float32,
    )
    z = jnp.tanh(y + b_ref[...])
    for g in range(RB // HALF):
        zg = z[g * HALF:(g + 1) * HALF, :]
        o_ref[pl.ds(2 * g * HALF, HALF), :] = zg[:, :F]
        o_ref[pl.ds((2 * g + 1) * HALF, HALF), :] = zg[:, F:]


def _dense_body_chain(prev_ref, x_ref, w_ref, b_ref, o_ref):
    del prev_ref
    _dense_body_first(x_ref, w_ref, b_ref, o_ref)


@functools.lru_cache(maxsize=None)
def _make_dense(tok, base, nblk_seg, first):
    out_spec = pl.BlockSpec((2 * RB, F), lambda i: (base + i, 0))
    x_spec = pl.BlockSpec((RB, 2 * D), lambda i: (i, 0))
    w_spec = pl.BlockSpec((2 * D, 2 * F), lambda i: (0, 0))
    b_spec = pl.BlockSpec((1, 2 * F), lambda i: (0, 0))
    out_shape = jax.ShapeDtypeStruct((tok, F), jnp.float32)
    params = pltpu.CompilerParams(vmem_limit_bytes=60000 * 1024)
    if first:
        return pl.pallas_call(
            _dense_body_first,
            grid=(nblk_seg,),
            in_specs=[x_spec, w_spec, b_spec],
            out_specs=out_spec,
            out_shape=out_shape,
            compiler_params=params,
        )
    return pl.pallas_call(
        _dense_body_chain,
        grid=(nblk_seg,),
        in_specs=[
            pl.BlockSpec(memory_space=pl.ANY),
            x_spec, w_spec, b_spec,
        ],
        out_specs=out_spec,
        out_shape=out_shape,
        input_output_aliases={0: 0},
        compiler_params=params,
    )


def kernel(inputs, table, W, b):
    B, L = inputs.shape
    tok = B * L
    unit = NW * CHUNK * 2
    segs = [unit] + [(tok - unit) // (NSEG - 1)] * (NSEG - 1)
    idx = inputs.reshape(tok).astype(jnp.int32)
    zeros = jnp.zeros((D, F), jnp.float32)
    w_big = jnp.concatenate(
        [
            jnp.concatenate([W, zeros], axis=1),
            jnp.concatenate([zeros, W], axis=1),
        ],
        axis=0,
    )
    w_big = w_big.astype(jnp.bfloat16)
    b2 = jnp.concatenate([b, b]).reshape(1, 2 * F)
    embs = []
    off = 0
    for seg_tok in segs:
        embs.append(
            (_make_gather(seg_tok)(
                jax.lax.slice(idx, (off,), (off + seg_tok,)), table
            ), off, seg_tok)
        )
        off += seg_tok
    out = None
    for s, (emb, off, seg_tok) in enumerate(embs):
        emb2 = emb.reshape(seg_tok // 2, 2 * D)
        base = off // (2 * RB)
        nblk = seg_tok // (2 * RB)
        if s == 0:
            out = _make_dense(tok, base, nblk, True)(emb2, w_big, b2)
        else:
            out = _make_dense(tok, base, nblk, False)(out, emb2, w_big, b2)
    return out.reshape(B, L, F)


# speedup vs baseline: 1.0036x; 1.0036x over previous
import functools

import jax
import jax.numpy as jnp
from jax import lax
from jax.experimental import pallas as pl
from jax.experimental.pallas import tpu as pltpu
from jax.experimental.pallas import tpu_sc as plsc

D = 64
F = 128

NC = 2
NS = 16
NW = NC * NS

IDX_ROW = 128
SUBS = 4
CHUNK = IDX_ROW * SUBS
HALF = CHUNK // 2


def _gather_body(idx_hbm, table_hbm, out_hbm, raw_v, idx_v, rows_v, gsem0,
                 gsem1, osem0, osem1, *, n_chunks):
    wid = lax.axis_index("s") * NC + lax.axis_index("c")
    base_tok = wid * (n_chunks * CHUNK)
    ar = jax.lax.iota(jnp.int32, 16)
    pbase = (ar // 2) + (ar % 2) * HALF
    gsems = [gsem0, gsem1]
    osems = [osem0, osem1]

    def stage(c, b):
        tok_off = base_tok + c * CHUNK
        pltpu.sync_copy(idx_hbm.at[pl.ds(tok_off, CHUNK)], raw_v.at[b])
        for g in range(CHUNK // 16):
            vals = plsc.load_gather(raw_v.at[b], [pbase + (8 * g)])
            idx_v[b, (g * 16) // IDX_ROW, pl.ds((g * 16) % IDX_ROW, 16)] = vals
        for j in range(SUBS):
            pltpu.async_copy(
                table_hbm.at[idx_v.at[b].at[j]],
                rows_v.at[b].at[pl.ds(j * IDX_ROW, IDX_ROW)],
                gsems[b],
            )

    def drain_gathers(b):
        for j in range(SUBS):
            pltpu.make_async_copy(
                table_hbm.at[idx_v.at[b].at[j]],
                rows_v.at[b].at[pl.ds(j * IDX_ROW, IDX_ROW)],
                gsems[b],
            ).wait()

    def flush(c, b):
        tok_off = base_tok + c * CHUNK
        pltpu.async_copy(
            rows_v.at[b], out_hbm.at[pl.ds(tok_off, CHUNK)], osems[b]
        )

    def drain_flush(c, b):
        tok_off = base_tok + c * CHUNK
        pltpu.make_async_copy(
            rows_v.at[b], out_hbm.at[pl.ds(tok_off, CHUNK)], osems[b]
        ).wait()

    stage(0, 0)
    np2 = n_chunks // 2

    def body(p, carry):
        c0 = 2 * p

        stage(c0 + 1, 1)
        drain_gathers(0)
        flush(c0, 0)
        drain_gathers(1)
        drain_flush(c0, 0)

        @pl.when(p + 1 < np2)
        def _():
            stage(c0 + 2, 0)

        flush(c0 + 1, 1)
        drain_flush(c0 + 1, 1)
        return carry

    lax.fori_loop(0, np2, body, 0)


@functools.lru_cache(maxsize=None)
def _make_gather(tok):
    n_chunks = tok // (NW * CHUNK)
    mesh = plsc.VectorSubcoreMesh(core_axis_name="c", subcore_axis_name="s")
    return pl.kernel(
        functools.partial(_gather_body, n_chunks=n_chunks),
        out_type=jax.ShapeDtypeStruct((tok, D), jnp.float32),
        mesh=mesh,
        scratch_types=[
            pltpu.VMEM((2, CHUNK), jnp.int32),
            pltpu.VMEM((2, SUBS, IDX_ROW), jnp.int32),
            pltpu.VMEM((2, CHUNK, D), jnp.float32),
            pltpu.SemaphoreType.DMA,
            pltpu.SemaphoreType.DMA,
            pltpu.SemaphoreType.DMA,
            pltpu.SemaphoreType.DMA,
        ],
        compiler_params=pltpu.CompilerParams(
            use_tc_tiling_on_sc=False, needs_layout_passes=False
        ),
    )


RB = 8192


NSEG = 5


def _dense_body_first(x_ref, w_ref, b_ref, o_ref):
    y = jnp.dot(
        x_ref[...].astype(jnp.bfloat16),
        w_ref[...],
        preferred_element_type=jnp.float32,
    )
    z = jnp.tanh(y + b_ref[...])
    for g in range(RB // HALF):
        zg = z[g * HALF:(g + 1) * HALF, :]
        o_ref[pl.ds(2 * g * HALF, HALF), :] = zg[:, :F]
        o_ref[pl.ds((2 * g + 1) * HALF, HALF), :] = zg[:, F:]


def _dense_body_chain(prev_ref, x_ref, w_ref, b_ref, o_ref):
    del prev_ref
    _dense_body_first(x_ref, w_ref, b_ref, o_ref)


@functools.lru_cache(maxsize=None)
def _make_dense(tok, seg, first):
    nblk_seg = tok // NSEG // (2 * RB)
    base = seg * nblk_seg
    out_spec = pl.BlockSpec((2 * RB, F), lambda i: (base + i, 0))
    x_spec = pl.BlockSpec((RB, 2 * D), lambda i: (i, 0))
    w_spec = pl.BlockSpec((2 * D, 2 * F), lambda i: (0, 0))
    b_spec = pl.BlockSpec((1, 2 * F), lambda i: (0, 0))
    out_shape = jax.ShapeDtypeStruct((tok, F), jnp.float32)
    params = pltpu.CompilerParams(vmem_limit_bytes=60000 * 1024)
    if first:
        return pl.pallas_call(
            _dense_body_first,
            grid=(nblk_seg,),
            in_specs=[x_spec, w_spec, b_spec],
            out_specs=out_spec,
            out_shape=out_shape,
            compiler_params=params,
        )
    return pl.pallas_call(
        _dense_body_chain,
        grid=(nblk_seg,),
        in_specs=[
            pl.BlockSpec(memory_space=pl.ANY),
            x_spec, w_spec, b_spec,
        ],
        out_specs=out_spec,
        out_shape=out_shape,
        input_output_aliases={0: 0},
        compiler_params=params,
    )


def kernel(inputs, table, W, b):
    B, L = inputs.shape
    tok = B * L
    tok_seg = tok // NSEG
    idx = inputs.reshape(tok).astype(jnp.int32)
    zeros = jnp.zeros((D, F), jnp.float32)
    w_big = jnp.concatenate(
        [
            jnp.concatenate([W, zeros], axis=1),
            jnp.concatenate([zeros, W], axis=1),
        ],
        axis=0,
    )
    w_big = w_big.astype(jnp.bfloat16)
    b2 = jnp.concatenate([b, b]).reshape(1, 2 * F)
    gather = _make_gather(tok_seg)
    embs = [
        gather(jax.lax.slice(idx, (s * tok_seg,), ((s + 1) * tok_seg,)), table)
        for s in range(NSEG)
    ]
    out = None
    for s in range(NSEG):
        emb2 = embs[s].reshape(tok_seg // 2, 2 * D)
        if s == 0:
            out = _make_dense(tok, 0, True)(emb2, w_big, b2)
        else:
            out = _make_dense(tok, s, False)(out, emb2, w_big, b2)
    return out.reshape(B, L, F)
